# baseline (speedup 1.0000x reference)
import functools

import jax
import jax.numpy as jnp
from jax import lax
from jax.experimental import pallas as pl
from jax.experimental.pallas import tpu as pltpu
from jax.experimental.pallas import tpu_sc as plsc

NUM_EMBEDDINGS = 100000
PACKED_WORDS = 16
EMB_DIM = 128
BATCH = 16384
FIELDS = 26
ROWS = BATCH * FIELDS
NC, NS, L = 2, 16, 16
NW = NC * NS
ROWS_PER_W = ROWS // NW
CHUNK = 128
NCHUNK = ROWS_PER_W // CHUNK
NBUF = 4
TBLK = 1000


def _dequant_table_body(tab_ref, c_ref, out_ref):
  w = tab_ref[...]
  inv = 1.0 / c_ref[0, 0]
  src = jnp.broadcast_to(w[:, :, None], (TBLK, PACKED_WORDS, 8))
  src = src.reshape(TBLK, EMB_DIM)
  lane = lax.broadcasted_iota(jnp.int32, (TBLK, EMB_DIM), 1)
  shift = 8 * ((lane // 2) & 3) + 4 * (1 - (lane & 1))
  nib = lax.shift_right_logical(src, shift) & 15
  out_ref[...] = (nib.astype(jnp.float32) - 7.0) * inv


_dequant_table = pl.pallas_call(
    _dequant_table_body,
    grid=(NUM_EMBEDDINGS // TBLK,),
    in_specs=[
        pl.BlockSpec((TBLK, PACKED_WORDS), lambda i: (i, 0)),
        pl.BlockSpec(memory_space=pltpu.SMEM),
    ],
    out_specs=pl.BlockSpec((TBLK, EMB_DIM), lambda i: (i, 0)),
    out_shape=jax.ShapeDtypeStruct((NUM_EMBEDDINGS, EMB_DIM), jnp.float32),
)


def _make_gather_kernel():
  mesh = plsc.VectorSubcoreMesh(core_axis_name="c", subcore_axis_name="s")

  @functools.partial(
      pl.kernel,
      mesh=mesh,
      out_type=jax.ShapeDtypeStruct((ROWS, EMB_DIM), jnp.float32),
      compiler_params=pltpu.CompilerParams(use_tc_tiling_on_sc=False),
      scratch_types=[
          pltpu.VMEM((ROWS_PER_W,), jnp.int32),
          pltpu.VMEM((CHUNK, EMB_DIM), jnp.float32),
          pltpu.VMEM((CHUNK, EMB_DIM), jnp.float32),
          pltpu.VMEM((CHUNK, EMB_DIM), jnp.float32),
          pltpu.VMEM((CHUNK, EMB_DIM), jnp.float32),
          pltpu.SemaphoreType.DMA,
          pltpu.SemaphoreType.DMA,
          pltpu.SemaphoreType.DMA,
          pltpu.SemaphoreType.DMA,
          pltpu.SemaphoreType.DMA,
          pltpu.SemaphoreType.DMA,
          pltpu.SemaphoreType.DMA,
          pltpu.SemaphoreType.DMA,
      ],
  )
  def k(tab_hbm, idx_hbm, out_hbm, idx_all, b0, b1, b2, b3,
        sg0, sg1, sg2, sg3, so0, so1, so2, so3):
    wid = lax.axis_index("s") * NC + lax.axis_index("c")
    tbase = wid * ROWS_PER_W

    bufs = (b0, b1, b2, b3)
    sg = (sg0, sg1, sg2, sg3)
    so = (so0, so1, so2, so3)

    pltpu.sync_copy(idx_hbm.at[pl.ds(tbase, ROWS_PER_W)], idx_all)

    def start_gather(g, b):
      pltpu.async_copy(
          tab_hbm.at[idx_all.at[pl.ds(g * CHUNK, CHUNK)]], bufs[b], sg[b])

    def wait_gather(b):
      pltpu.make_async_copy(
          tab_hbm.at[pl.ds(0, CHUNK), :], bufs[b], sg[b]).wait()

    def start_out(g, b):
      pltpu.async_copy(
          bufs[b], out_hbm.at[pl.ds(tbase + g * CHUNK, CHUNK), :], so[b])

    def wait_out(b):
      pltpu.make_async_copy(
          bufs[b], out_hbm.at[pl.ds(0, CHUNK), :], so[b]).wait()

    for b in range(NBUF - 1):
      start_gather(b, b)

    def quad(q, _):
      for b in range(NBUF):
        g = NBUF * q + b
        nxt = g + NBUF - 1
        nb = (b + NBUF - 1) % NBUF

        wait_gather(b)
        start_out(g, b)

        @pl.when(jnp.logical_and(nxt >= NBUF, nxt < NCHUNK))
        def _():
          wait_out(nb)

        @pl.when(nxt < NCHUNK)
        def _():
          start_gather(nxt, nb)
      return 0

    lax.fori_loop(0, NCHUNK // NBUF, quad, 0)
    for b in range(NBUF):
      wait_out(b)

  return k


_gather_kernel = _make_gather_kernel()


@jax.jit
def kernel(x, weight_quant_packed, c):
  idx = x.T.reshape(ROWS)
  tab32 = lax.bitcast_convert_type(
      weight_quant_packed.reshape(NUM_EMBEDDINGS, PACKED_WORDS, 4), jnp.int32)
  tabq = _dequant_table(tab32, jnp.reshape(c, (1, 1)))
  out = _gather_kernel(tabq, idx)
  return out.reshape(FIELDS, BATCH, EMB_DIM).transpose(1, 0, 2)


# speedup vs baseline: 2.4408x; 2.4408x over previous
import functools

import jax
import jax.numpy as jnp
from jax import lax
from jax.experimental import pallas as pl
from jax.experimental.pallas import tpu as pltpu
from jax.experimental.pallas import tpu_sc as plsc

NUM_EMBEDDINGS = 100000
PACKED_WORDS = 16
EMB_DIM = 128
BATCH = 16384
FIELDS = 26
ROWS = BATCH * FIELDS
NC, NS, L = 2, 16, 16
NW = NC * NS
ROWS_PER_W = ROWS // NW
CHUNK = 128
NCHUNK = ROWS_PER_W // CHUNK


def _make_kernel():
  mesh = plsc.VectorSubcoreMesh(core_axis_name="c", subcore_axis_name="s")

  @functools.partial(
      pl.kernel,
      mesh=mesh,
      out_type=jax.ShapeDtypeStruct((ROWS, EMB_DIM), jnp.float32),
      compiler_params=pltpu.CompilerParams(use_tc_tiling_on_sc=False),
      scratch_types=[
          pltpu.VMEM((ROWS_PER_W,), jnp.int32),
          pltpu.VMEM((CHUNK, PACKED_WORDS), jnp.int32),
          pltpu.VMEM((CHUNK, PACKED_WORDS), jnp.int32),
          pltpu.VMEM((CHUNK, EMB_DIM), jnp.float32),
          pltpu.VMEM((CHUNK, EMB_DIM), jnp.float32),
          pltpu.VMEM((L,), jnp.float32),
          pltpu.SemaphoreType.DMA,
          pltpu.SemaphoreType.DMA,
          pltpu.SemaphoreType.DMA,
          pltpu.SemaphoreType.DMA,
      ],
  )
  def k(tab_hbm, idx_hbm, c_hbm, out_hbm, idx_all, g0, g1, o0, o1,
        c_v, sg0, sg1, so0, so1):
    wid = lax.axis_index("s") * NC + lax.axis_index("c")
    tbase = wid * ROWS_PER_W

    gb = (g0, g1)
    ob = (o0, o1)
    sg = (sg0, sg1)
    so = (so0, so1)

    pltpu.sync_copy(idx_hbm.at[pl.ds(tbase, ROWS_PER_W)], idx_all)
    pltpu.sync_copy(c_hbm, c_v)
    lut = (lax.iota(jnp.int32, L).astype(jnp.float32) - 7.0) / c_v[...]

    it = lax.iota(jnp.int32, L)
    wordsel = it >> 3
    shvec = ((it >> 1) & 3) * 8 + (1 - (it & 1)) * 4

    def vperm(src, idx):
      return lax.gather(
          src, idx[:, None],
          lax.GatherDimensionNumbers(
              offset_dims=(), collapsed_slice_dims=(0,),
              start_index_map=(0,)),
          slice_sizes=(1,),
          mode=lax.GatherScatterMode.PROMISE_IN_BOUNDS)

    def start_gather(g, b):
      pltpu.async_copy(
          tab_hbm.at[idx_all.at[pl.ds(g * CHUNK, CHUNK)]], gb[b], sg[b])

    def wait_gather(b):
      pltpu.make_async_copy(
          tab_hbm.at[pl.ds(0, CHUNK), :], gb[b], sg[b]).wait()

    def start_out(g, b):
      pltpu.async_copy(
          ob[b], out_hbm.at[pl.ds(tbase + g * CHUNK, CHUNK), :], so[b])

    def wait_out(b):
      pltpu.make_async_copy(
          ob[b], out_hbm.at[pl.ds(0, CHUNK), :], so[b]).wait()

    start_gather(0, 0)

    def chunk_pair(g2, _):
      for b in range(2):
        g = 2 * g2 + b
        nxt = g + 1

        @pl.when(nxt < NCHUNK)
        def _():
          start_gather(nxt, 1 - b)

        wait_gather(b)

        @pl.when(g >= 2)
        def _():
          wait_out(b)

        gbb = gb[b]
        obb = ob[b]

        def row_body(i, _):
          w = gbb[i, :]
          for s in range(8):
            ws = vperm(w, wordsel + 2 * s)
            nib = lax.shift_right_logical(ws, shvec) & 15
            obb[i, pl.ds(s * L, L)] = vperm(lut, nib)
          return 0

        lax.fori_loop(0, CHUNK, row_body, 0, unroll=4)
        start_out(g, b)
      return 0

    lax.fori_loop(0, NCHUNK // 2, chunk_pair, 0)
    wait_out(0)
    wait_out(1)

  return k


_sc_kernel = _make_kernel()


@jax.jit
def kernel(x, weight_quant_packed, c):
  idx = x.T.reshape(ROWS)
  c_vec = jnp.full((L,), c, dtype=jnp.float32)
  tab32 = lax.bitcast_convert_type(
      weight_quant_packed.reshape(NUM_EMBEDDINGS, PACKED_WORDS, 4), jnp.int32)
  out = _sc_kernel(tab32, idx, c_vec)
  return out.reshape(FIELDS, BATCH, EMB_DIM).transpose(1, 0, 2)
